# baseline (speedup 1.0000x reference)
import functools

import jax
import jax.numpy as jnp
from jax import lax
from jax.experimental import pallas as pl
from jax.experimental.pallas import tpu as pltpu
from jax.experimental.pallas import tpu_sc as plsc

NHEADS = 16
MAXDIST = 128
NBUCKETS = 2 * MAXDIST - 1
QL = 2048
KL = 2048

NSHIFT = 8
WCOLS = 3072
ROWS_PER_TILE = QL // 2
GRP = 8
GROUPS_PER_SHIFT = ROWS_PER_TILE // NSHIFT // GRP
LAG_GRPS = 8


def _body(bias_hbm, off_hbm, out_hbm, bias_v, off_v, wbuf_v, sem):
    cid = lax.axis_index("c")
    sid = lax.axis_index("s")
    wid = sid * 2 + cid
    h = wid >> 1
    half = wid & 1
    row0 = half * ROWS_PER_TILE
    wstart = QL // 2 - row0

    pltpu.sync_copy(bias_hbm, bias_v)
    pltpu.sync_copy(off_hbm, off_v)

    off16 = off_v[...]
    h16 = jnp.full((16,), h, dtype=jnp.int32)
    iota16 = lax.broadcasted_iota(jnp.int32, (16,), 0)
    cbase = (wstart - (QL - 1) + (MAXDIST - 1)) + off16

    def drain_grp():
        pltpu.make_async_copy(
            wbuf_v.at[pl.ds(0, GRP * KL)],
            out_hbm.at[pl.ds(row0 * KL, GRP * KL)],
            sem,
        ).wait()

    for r in range(NSHIFT):
        def build(k, carry, r=r):
            u = k * 16 + iota16
            c = jnp.clip(u + (cbase + (NSHIFT - 1 - r)), 0, NBUCKETS - 1)
            vals = plsc.load_gather(bias_v, [c * NHEADS + h16])
            wbuf_v[pl.ds(r * WCOLS + k * 16, 16)] = vals
            return carry

        lax.fori_loop(0, WCOLS // 16, build, 0)

        def fire(g, carry, r=r):
            i0 = row0 + (GRP * NSHIFT) * g + r
            u00 = r * WCOLS + 1016 - (GRP * NSHIFT) * g
            d00 = (h * QL + i0) * KL
            for t in range(GRP):
                u0 = pl.multiple_of(u00 - NSHIFT * t, 8)
                dst0 = pl.multiple_of(d00 + (NSHIFT * t) * KL, 8)
                for half2 in range(2):
                    pltpu.make_async_copy(
                        wbuf_v.at[pl.ds(u0 + half2 * (KL // 2), KL // 2)],
                        out_hbm.at[pl.ds(dst0 + half2 * (KL // 2), KL // 2)],
                        sem,
                    ).start()

            @pl.when(r * (GROUPS_PER_SHIFT) + g >= LAG_GRPS)
            def _():
                drain_grp()

            return carry

        lax.fori_loop(0, GROUPS_PER_SHIFT, fire, 0)

    def drain(_, carry):
        drain_grp()
        return carry

    lax.fori_loop(0, LAG_GRPS, drain, 0)


def kernel(qlen, klen, bias):
    off = jnp.asarray(klen, jnp.int32) - jnp.asarray(qlen, jnp.int32)
    off_arr = jnp.full((16,), off, dtype=jnp.int32)
    mesh = plsc.VectorSubcoreMesh(core_axis_name="c", subcore_axis_name="s")
    run = functools.partial(
        pl.kernel,
        mesh=mesh,
        compiler_params=pltpu.CompilerParams(needs_layout_passes=False),
        out_type=jax.ShapeDtypeStruct((NHEADS * QL * KL,), jnp.float32),
        scratch_types=[
            pltpu.VMEM((NBUCKETS * NHEADS,), jnp.float32),
            pltpu.VMEM((16,), jnp.int32),
            pltpu.VMEM((NSHIFT * WCOLS,), jnp.float32),
            pltpu.SemaphoreType.DMA,
        ],
    )(_body)
    flat = run(bias.astype(jnp.float32).reshape(-1), off_arr)
    return flat.reshape(NHEADS, QL, KL)


# speedup vs baseline: 3.3093x; 3.3093x over previous
import functools

import jax
import jax.numpy as jnp
from jax import lax
from jax.experimental import pallas as pl
from jax.experimental.pallas import tpu as pltpu
from jax.experimental.pallas import tpu_sc as plsc

NHEADS = 16
MAXDIST = 128
NBUCKETS = 2 * MAXDIST - 1
QL = 2048
KL = 2048

NSHIFT = 8
WCOLS = 3072
ROWS_PER_TILE = QL // 2
LAG = 64


def _body(bias_hbm, off_hbm, out_hbm, bias_v, off_v, wbuf_v, sem):
    cid = lax.axis_index("c")
    sid = lax.axis_index("s")
    wid = sid * 2 + cid
    h = wid >> 1
    half = wid & 1
    row0 = half * ROWS_PER_TILE
    wstart = QL // 2 - row0

    pltpu.sync_copy(bias_hbm, bias_v)
    pltpu.sync_copy(off_hbm, off_v)

    off16 = off_v[...]
    h16 = jnp.full((16,), h, dtype=jnp.int32)
    iota16 = lax.broadcasted_iota(jnp.int32, (16,), 0)
    cbase = (wstart - (QL - 1) + (MAXDIST - 1)) + off16

    def drain_piece():
        pltpu.make_async_copy(
            wbuf_v.at[pl.ds(0, 128)], out_hbm.at[pl.ds(row0 * KL, 128)], sem
        ).wait()

    for r in range(NSHIFT):
        def build(k, carry, r=r):
            u = k * 16 + iota16
            c = jnp.clip(u + (cbase + (NSHIFT - 1 - r)), 0, NBUCKETS - 1)
            vals = plsc.load_gather(bias_v, [c * NHEADS + h16])
            wbuf_v[pl.ds(r * WCOLS + k * 16, 16)] = vals
            return carry

        lax.fori_loop(0, WCOLS // 16, build, 0)

    def fire(k, carry):
        ql = k >> 4
        jt = k & 15
        u0 = 1016 - 8 * ql + 128 * jt
        dbase = (h * 256 + (row0 // 8 + ql)) * 16384 + jt * 1024
        for s in range(NSHIFT):
            pltpu.make_async_copy(
                wbuf_v.at[pl.ds(pl.multiple_of(s * WCOLS + u0, 8), 128)],
                out_hbm.at[pl.ds(pl.multiple_of(dbase + s * 128, 128), 128)],
                sem,
            ).start()

        @pl.when(k >= LAG)
        def _():
            for _s in range(NSHIFT):
                drain_piece()

        return carry

    lax.fori_loop(0, (ROWS_PER_TILE // NSHIFT) * 16, fire, 0)

    def drain(_, carry):
        for _s in range(NSHIFT):
            drain_piece()
        return carry

    lax.fori_loop(0, LAG, drain, 0)


def kernel(qlen, klen, bias):
    off = jnp.asarray(klen, jnp.int32) - jnp.asarray(qlen, jnp.int32)
    off_arr = jnp.full((16,), off, dtype=jnp.int32)
    mesh = plsc.VectorSubcoreMesh(core_axis_name="c", subcore_axis_name="s")
    run = functools.partial(
        pl.kernel,
        mesh=mesh,
        compiler_params=pltpu.CompilerParams(needs_layout_passes=False),
        out_type=jax.ShapeDtypeStruct((NHEADS * QL * KL,), jnp.float32),
        scratch_types=[
            pltpu.VMEM((NBUCKETS * NHEADS,), jnp.float32),
            pltpu.VMEM((16,), jnp.int32),
            pltpu.VMEM((NSHIFT * WCOLS,), jnp.float32),
            pltpu.SemaphoreType.DMA,
        ],
    )(_body)
    flat = run(bias.astype(jnp.float32).reshape(-1), off_arr)
    out5 = flat.reshape(NHEADS, QL // 8, KL // 128, 8, 128)
    return out5.transpose(0, 1, 3, 2, 4).reshape(NHEADS, QL, KL)


# speedup vs baseline: 3.3215x; 1.0037x over previous
import functools

import jax
import jax.numpy as jnp
from jax import lax
from jax.experimental import pallas as pl
from jax.experimental.pallas import tpu as pltpu
from jax.experimental.pallas import tpu_sc as plsc

NHEADS = 16
MAXDIST = 128
NBUCKETS = 2 * MAXDIST - 1
QL = 2048
KL = 2048

NSHIFT = 8
WCOLS = 3072
ROWS_PER_TILE = QL // 2
LAG = 64


def _body(bias_hbm, off_hbm, out_hbm, bias_v, off_v, wbuf_v, sem):
    cid = lax.axis_index("c")
    sid = lax.axis_index("s")
    wid = sid * 2 + cid
    h = wid >> 1
    half = wid & 1
    row0 = half * ROWS_PER_TILE
    wstart = QL // 2 - row0

    pltpu.sync_copy(bias_hbm, bias_v)
    pltpu.sync_copy(off_hbm, off_v)

    off16 = off_v[...]
    h16 = jnp.full((16,), h, dtype=jnp.int32)
    iota16 = lax.broadcasted_iota(jnp.int32, (16,), 0)
    cbase = (wstart - (QL - 1) + (MAXDIST - 1)) + off16

    def drain_iter():
        pltpu.make_async_copy(
            wbuf_v.at[pl.ds(0, 1024)], out_hbm.at[pl.ds(row0 * KL, 1024)], sem
        ).wait()

    for r in range(NSHIFT):
        def build(k, carry, r=r):
            u = k * 16 + iota16
            c = jnp.clip(u + (cbase + (NSHIFT - 1 - r)), 0, NBUCKETS - 1)
            vals = plsc.load_gather(bias_v, [c * NHEADS + h16])
            wbuf_v[pl.ds(r * WCOLS + k * 16, 16)] = vals
            return carry

        lax.fori_loop(0, WCOLS // 16, build, 0)

    def fire(k, carry):
        ql = k >> 4
        jt = k & 15
        u0 = 1016 - 8 * ql + 128 * jt
        dbase = (h * 256 + (row0 // 8 + ql)) * 16384 + jt * 1024
        for s in range(NSHIFT):
            pltpu.make_async_copy(
                wbuf_v.at[pl.ds(pl.multiple_of(s * WCOLS + u0, 8), 128)],
                out_hbm.at[pl.ds(pl.multiple_of(dbase + s * 128, 128), 128)],
                sem,
            ).start()

        @pl.when(k >= LAG)
        def _():
            drain_iter()

        return carry

    lax.fori_loop(0, (ROWS_PER_TILE // NSHIFT) * 16, fire, 0)

    def drain(_, carry):
        drain_iter()
        return carry

    lax.fori_loop(0, LAG, drain, 0)


def kernel(qlen, klen, bias):
    off = jnp.asarray(klen, jnp.int32) - jnp.asarray(qlen, jnp.int32)
    off_arr = jnp.full((16,), off, dtype=jnp.int32)
    mesh = plsc.VectorSubcoreMesh(core_axis_name="c", subcore_axis_name="s")
    run = functools.partial(
        pl.kernel,
        mesh=mesh,
        compiler_params=pltpu.CompilerParams(needs_layout_passes=False),
        out_type=jax.ShapeDtypeStruct((NHEADS * QL * KL,), jnp.float32),
        scratch_types=[
            pltpu.VMEM((NBUCKETS * NHEADS,), jnp.float32),
            pltpu.VMEM((16,), jnp.int32),
            pltpu.VMEM((NSHIFT * WCOLS,), jnp.float32),
            pltpu.SemaphoreType.DMA,
        ],
    )(_body)
    flat = run(bias.astype(jnp.float32).reshape(-1), off_arr)
    out5 = flat.reshape(NHEADS, QL // 8, KL // 128, 8, 128)
    return out5.transpose(0, 1, 3, 2, 4).reshape(NHEADS, QL, KL)


# speedup vs baseline: 3.4058x; 1.0254x over previous
import functools

import jax
import jax.numpy as jnp
from jax import lax
from jax.experimental import pallas as pl
from jax.experimental.pallas import tpu as pltpu
from jax.experimental.pallas import tpu_sc as plsc

NHEADS = 16
MAXDIST = 128
NBUCKETS = 2 * MAXDIST - 1
QL = 2048
KL = 2048

NSHIFT = 8
WCOLS = 3072
ROWS_PER_TILE = QL // 2
LAG = 64


def _body(bias_hbm, off_hbm, out_hbm, bias_v, off_v, wbuf_v, sem):
    cid = lax.axis_index("c")
    sid = lax.axis_index("s")
    wid = sid * 2 + cid
    h = wid >> 1
    half = wid & 1
    row0 = half * ROWS_PER_TILE
    wstart = QL // 2 - row0

    pltpu.sync_copy(bias_hbm, bias_v)
    pltpu.sync_copy(off_hbm, off_v)

    off16 = off_v[...]
    h16 = jnp.full((16,), h, dtype=jnp.int32)
    iota16 = lax.broadcasted_iota(jnp.int32, (16,), 0)
    cbase = (wstart - (QL - 1) + (MAXDIST - 1)) + off16

    def drain_iter():
        pltpu.make_async_copy(
            wbuf_v.at[pl.ds(0, 1024)], out_hbm.at[pl.ds(row0 * KL, 1024)], sem
        ).wait()

    for s in range(NSHIFT):
        def build(k, carry, s=s):
            u = k * 16 + iota16
            c = jnp.clip(u + (cbase + (NSHIFT - 1 - s)), 0, NBUCKETS - 1)
            vals = plsc.load_gather(bias_v, [c * NHEADS + h16])
            wbuf_v[pl.ds(s * WCOLS + k * 16, 16)] = vals
            return carry

        lax.fori_loop(0, WCOLS // 16, build, 0)

        def fire(k2, carry, s=s):
            ql = k2 >> 1
            jth = k2 & 1
            u0 = 1016 - 8 * ql + 128 * (8 * jth)
            dbase = (h * 256 + (row0 // 8 + ql)) * 16384 + (8 * jth) * 1024 + s * 128
            for jtl in range(8):
                pltpu.make_async_copy(
                    wbuf_v.at[pl.ds(pl.multiple_of(s * WCOLS + u0 + 128 * jtl, 8), 128)],
                    out_hbm.at[pl.ds(pl.multiple_of(dbase + jtl * 1024, 128), 128)],
                    sem,
                ).start()

            @pl.when(s * (2 * (ROWS_PER_TILE // NSHIFT)) + k2 >= LAG)
            def _():
                drain_iter()

            return carry

        lax.fori_loop(0, 2 * (ROWS_PER_TILE // NSHIFT), fire, 0)

    def drain(_, carry):
        drain_iter()
        return carry

    lax.fori_loop(0, LAG, drain, 0)


def kernel(qlen, klen, bias):
    off = jnp.asarray(klen, jnp.int32) - jnp.asarray(qlen, jnp.int32)
    off_arr = jnp.full((16,), off, dtype=jnp.int32)
    mesh = plsc.VectorSubcoreMesh(core_axis_name="c", subcore_axis_name="s")
    run = functools.partial(
        pl.kernel,
        mesh=mesh,
        compiler_params=pltpu.CompilerParams(needs_layout_passes=False),
        out_type=jax.ShapeDtypeStruct((NHEADS * QL * KL,), jnp.float32),
        scratch_types=[
            pltpu.VMEM((NBUCKETS * NHEADS,), jnp.float32),
            pltpu.VMEM((16,), jnp.int32),
            pltpu.VMEM((NSHIFT * WCOLS,), jnp.float32),
            pltpu.SemaphoreType.DMA,
        ],
    )(_body)
    flat = run(bias.astype(jnp.float32).reshape(-1), off_arr)
    out5 = flat.reshape(NHEADS, QL // 8, KL // 128, 8, 128)
    return out5.transpose(0, 1, 3, 2, 4).reshape(NHEADS, QL, KL)


# speedup vs baseline: 3.4127x; 1.0020x over previous
import functools

import jax
import jax.numpy as jnp
from jax import lax
from jax.experimental import pallas as pl
from jax.experimental.pallas import tpu as pltpu
from jax.experimental.pallas import tpu_sc as plsc

NHEADS = 16
MAXDIST = 128
NBUCKETS = 2 * MAXDIST - 1
QL = 2048
KL = 2048

NSHIFT = 8
WCOLS = 3072
ROWS_PER_TILE = QL // 2
LAG = 128
DRAIN_EVERY = 4


def _body(bias_hbm, off_hbm, out_hbm, bias_v, off_v, wbuf_v, sem):
    cid = lax.axis_index("c")
    sid = lax.axis_index("s")
    wid = sid * 2 + cid
    h = wid >> 1
    half = wid & 1
    row0 = half * ROWS_PER_TILE
    wstart = QL // 2 - row0

    pltpu.sync_copy(bias_hbm, bias_v)
    pltpu.sync_copy(off_hbm, off_v)

    off16 = off_v[...]
    h16 = jnp.full((16,), h, dtype=jnp.int32)
    iota16 = lax.broadcasted_iota(jnp.int32, (16,), 0)
    cbase = (wstart - (QL - 1) + (MAXDIST - 1)) + off16

    def drain_batch():
        pltpu.make_async_copy(
            wbuf_v.at[pl.ds(0, DRAIN_EVERY * 1024)],
            out_hbm.at[pl.ds(row0 * KL, DRAIN_EVERY * 1024)],
            sem,
        ).wait()

    for s in range(NSHIFT):
        def build(k, carry, s=s):
            u = k * 16 + iota16
            c = jnp.clip(u + (cbase + (NSHIFT - 1 - s)), 0, NBUCKETS - 1)
            vals = plsc.load_gather(bias_v, [c * NHEADS + h16])
            wbuf_v[pl.ds(s * WCOLS + k * 16, 16)] = vals
            return carry

        lax.fori_loop(0, WCOLS // 16, build, 0)

        def fire(k2, carry, s=s):
            ql = k2 >> 1
            jth = k2 & 1
            u0 = 1016 - 8 * ql + 128 * (8 * jth)
            dbase = (h * 256 + (row0 // 8 + ql)) * 16384 + (8 * jth) * 1024 + s * 128
            for jtl in range(8):
                pltpu.make_async_copy(
                    wbuf_v.at[pl.ds(pl.multiple_of(s * WCOLS + u0 + 128 * jtl, 8), 128)],
                    out_hbm.at[pl.ds(pl.multiple_of(dbase + jtl * 1024, 128), 128)],
                    sem,
                ).start()

            g = s * (2 * (ROWS_PER_TILE // NSHIFT)) + k2

            @pl.when((g >= LAG) & ((g & (DRAIN_EVERY - 1)) == (DRAIN_EVERY - 1)))
            def _():
                drain_batch()

            return carry

        lax.fori_loop(0, 2 * (ROWS_PER_TILE // NSHIFT), fire, 0)

    def drain(_, carry):
        drain_batch()
        return carry

    lax.fori_loop(0, LAG // DRAIN_EVERY, drain, 0)


def kernel(qlen, klen, bias):
    off = jnp.asarray(klen, jnp.int32) - jnp.asarray(qlen, jnp.int32)
    off_arr = jnp.full((16,), off, dtype=jnp.int32)
    mesh = plsc.VectorSubcoreMesh(core_axis_name="c", subcore_axis_name="s")
    run = functools.partial(
        pl.kernel,
        mesh=mesh,
        compiler_params=pltpu.CompilerParams(needs_layout_passes=False),
        out_type=jax.ShapeDtypeStruct((NHEADS * QL * KL,), jnp.float32),
        scratch_types=[
            pltpu.VMEM((NBUCKETS * NHEADS,), jnp.float32),
            pltpu.VMEM((16,), jnp.int32),
            pltpu.VMEM((NSHIFT * WCOLS,), jnp.float32),
            pltpu.SemaphoreType.DMA,
        ],
    )(_body)
    flat = run(bias.astype(jnp.float32).reshape(-1), off_arr)
    out5 = flat.reshape(NHEADS, QL // 8, KL // 128, 8, 128)
    return out5.transpose(0, 1, 3, 2, 4).reshape(NHEADS, QL, KL)
